# baseline (speedup 1.0000x reference)
import functools

import jax
import jax.numpy as jnp
from jax import lax
from jax.experimental import pallas as pl
from jax.experimental.pallas import tpu as pltpu
from jax.experimental.pallas import tpu_sc as plsc

_B, _S, _D, _V, _NGRAM = 8, 512, 256, 384, 4
_BLOCKS = _NGRAM * (_NGRAM + 1) // 2
_NL = _BLOCKS - 1
_NSEG = 64
_HW = 128
_CH = 128
_NC = _S // _CH
_INTERPRET = False


def _pre_body(seq_ref, table_ref, convw_ref, convb_ref, projw_ref,
              projb_ref, emb_ref, ep_ref):
    f32 = jnp.float32
    S, D, V = _S, _D, _V
    ident = (lax.broadcasted_iota(jnp.int32, (S, S), 0)
             == lax.broadcasted_iota(jnp.int32, (S, S), 1)).astype(f32)
    seq_row = seq_ref[0].astype(f32)
    seq_col = lax.dot_general(ident, seq_row, (((1,), (1,)), ((), ())),
                              preferred_element_type=f32)
    vlanes = lax.broadcasted_iota(jnp.int32, (S, V), 1).astype(f32)
    onehot = (seq_col == vlanes).astype(f32)
    emb = jnp.dot(onehot, table_ref[...], preferred_element_type=f32)
    ep_ref[0:S, :] = emb
    ep_ref[S:S + 8, :] = jnp.zeros((8, D), f32)
    cw = convw_ref[...]
    acc = ep_ref[0:S, :] * cw[0:1, :]
    for tap in range(1, _NGRAM):
        acc += ep_ref[tap:tap + S, :] * cw[tap:tap + 1, :]
    acc += convb_ref[...]
    embed = lax.dot_general(acc, projw_ref[...], (((1,), (1,)), ((), ())),
                            preferred_element_type=f32) + projb_ref[...]
    emb_ref[0] = jnp.where(seq_col == 0.0, 0.0, embed)


def _tc_pre(seq3, table, convw2, convb2, proj_w, projb2):
    S, D, V = _S, _D, _V
    return pl.pallas_call(
        _pre_body,
        grid=(_B,),
        in_specs=[
            pl.BlockSpec((1, 1, S), lambda b: (b, 0, 0)),
            pl.BlockSpec((V, D), lambda b: (0, 0)),
            pl.BlockSpec((_NGRAM, D), lambda b: (0, 0)),
            pl.BlockSpec((1, D), lambda b: (0, 0)),
            pl.BlockSpec((D, D), lambda b: (0, 0)),
            pl.BlockSpec((1, D), lambda b: (0, 0)),
        ],
        out_specs=pl.BlockSpec((1, S, D), lambda b: (b, 0, 0)),
        out_shape=jax.ShapeDtypeStruct((_B, S, D), jnp.float32),
        scratch_shapes=[pltpu.VMEM((S + 8, D), jnp.float32)],
        compiler_params=pltpu.CompilerParams(
            dimension_semantics=("arbitrary",)),
        interpret=_INTERPRET,
    )(seq3, table, convw2, convb2, proj_w, projb2)


def _sc_segment_sums(embed, idx3, zacc):
    mesh = plsc.VectorSubcoreMesh(core_axis_name="c", subcore_axis_name="s")

    nrow = _NL * _NSEG

    @functools.partial(
        pl.kernel, mesh=mesh,
        out_type=jax.ShapeDtypeStruct((_B, 2, 2, nrow, _HW), jnp.float32),
        scratch_types=[
            pltpu.VMEM((_NL, _NC, _CH), jnp.int32),
            pltpu.VMEM_SHARED((16 * nrow, _HW), jnp.float32),
            pltpu.VMEM((_CH, _HW), jnp.float32),
            pltpu.SemaphoreType.DMA,
        ],
    )
    def k(e_hbm, idx_hbm, z_hbm, out_hbm, idx_v, acc_sh, ch_v, sem):
        s = lax.axis_index("s")
        wid = s * 2 + lax.axis_index("c")
        b = wid // 4
        h = (wid % 4) // 2
        r = wid % 2
        pltpu.sync_copy(idx_hbm.at[b], idx_v)
        off = jnp.broadcast_to(s * nrow, (16,)).astype(jnp.int32)
        for l in range(_NL):
            for cc in range(_NC):
                for kk in range(_CH // 16):
                    sl = pl.ds(kk * 16, 16)
                    idx_v[l, cc, sl] = idx_v[l, cc, sl] + off
        pltpu.sync_copy(z_hbm, acc_sh.at[pl.ds(s * nrow, nrow)])
        for c in range(2):
            pltpu.sync_copy(
                e_hbm.at[b, pl.ds((2 * r + c) * _CH, _CH),
                         pl.ds(h * _HW, _HW)], ch_v)
            cps = [pltpu.async_copy(ch_v, acc_sh.at[idx_v.at[l, 2 * r + c]],
                                    sem, add=True) for l in range(_NL)]
            for cp in cps:
                cp.wait()
        pltpu.sync_copy(acc_sh.at[pl.ds(s * nrow, nrow)],
                        out_hbm.at[b, h, r])

    return k(embed, idx3, zacc)


def _post_body(seq_ref, gid_ref, sums_ref, emb_ref, scorew_ref, ffw_ref,
               ffb_ref, out_ref, reps_ref):
    f32 = jnp.float32
    S, D = _S, _D
    ident = (lax.broadcasted_iota(jnp.int32, (S, S), 0)
             == lax.broadcasted_iota(jnp.int32, (S, S), 1)).astype(f32)

    def to_col(v_row):
        return lax.dot_general(ident, v_row, (((1,), (1,)), ((), ())),
                               preferred_element_type=f32)

    seq_col = to_col(seq_ref[0].astype(f32))
    embed = emb_ref[0]
    reps_ref[0] = embed

    def block_score(rep):
        return lax.dot_general(rep, scorew_ref[...], (((1,), (1,)), ((), ())),
                               preferred_element_type=f32)

    scores = [(block_score(embed), seq_col == 0.0)]

    glanes = lax.broadcasted_iota(jnp.int32, (S, _NSEG), 1).astype(f32)
    srows = lax.broadcasted_iota(jnp.int32, (S, _NSEG), 0).astype(f32)
    tri = (lax.broadcasted_iota(jnp.int32, (_NSEG, _NSEG), 0)
           < lax.broadcasted_iota(jnp.int32, (_NSEG, _NSEG), 1)).astype(f32)
    gl_row = lax.broadcasted_iota(jnp.int32, (1, _NSEG), 1).astype(f32)
    gid_all = gid_ref[0]
    sums_all = sums_ref[0]
    for l in range(_NL):
        g_row = gid_all[l:l + 1, :].astype(f32)
        g_col = to_col(g_row)
        gmax = jnp.max(g_row)
        is_pad = g_col == 0.0
        gadj = jnp.where(is_pad, gmax, g_col - 1.0)
        M = (gadj == glanes).astype(f32)
        counts = jnp.sum(M, axis=0, keepdims=True)
        recip = ((1.0 / jnp.maximum(counts, 1.0))
                 * (gl_row != gmax).astype(f32))
        cum = jnp.dot(counts, tri, preferred_element_type=f32)
        G = ((srows >= cum) & (srows < cum + counts)).astype(f32) * recip
        rsl = slice(l * _NSEG, (l + 1) * _NSEG)
        seg = jnp.concatenate(
            [sums_all[0, 0, rsl, :] + sums_all[0, 1, rsl, :],
             sums_all[1, 0, rsl, :] + sums_all[1, 1, rsl, :]],
            axis=1)
        rep = jnp.dot(G, seg, preferred_element_type=f32)
        reps_ref[l + 1] = rep
        scores.append((block_score(rep), is_pad))

    neg = -jnp.finfo(f32).max
    svals = [jnp.where(m, neg, s) for s, m in scores]
    mval = svals[0]
    for s in svals[1:]:
        mval = jnp.maximum(mval, s)
    exps = [jnp.exp(s - mval) for s in svals]
    den = exps[0]
    for e_ in exps[1:]:
        den = den + e_
    out = reps_ref[0] * (exps[0] / den)
    for l in range(1, _BLOCKS):
        out += reps_ref[l] * (exps[l] / den)
    y = lax.dot_general(out, ffw_ref[...], (((1,), (1,)), ((), ())),
                        preferred_element_type=f32) + ffb_ref[...]
    out_ref[0] = out + jnp.maximum(y, 0.0)


def _tc_post(seq3, group_id, sums, embed, score_w, ff_w, ffb2):
    S, D = _S, _D
    return pl.pallas_call(
        _post_body,
        grid=(_B,),
        in_specs=[
            pl.BlockSpec((1, 1, S), lambda b: (b, 0, 0)),
            pl.BlockSpec((1, _NL, S), lambda b: (b, 0, 0)),
            pl.BlockSpec((1, 2, 2, _NL * _NSEG, _HW),
                         lambda b: (b, 0, 0, 0, 0)),
            pl.BlockSpec((1, S, D), lambda b: (b, 0, 0)),
            pl.BlockSpec((1, D), lambda b: (0, 0)),
            pl.BlockSpec((D, D), lambda b: (0, 0)),
            pl.BlockSpec((1, D), lambda b: (0, 0)),
        ],
        out_specs=pl.BlockSpec((1, S, D), lambda b: (b, 0, 0)),
        out_shape=jax.ShapeDtypeStruct((_B, S, D), jnp.float32),
        scratch_shapes=[pltpu.VMEM((_BLOCKS, S, D), jnp.float32)],
        compiler_params=pltpu.CompilerParams(
            dimension_semantics=("arbitrary",)),
        interpret=_INTERPRET,
    )(seq3, group_id, sums, embed, score_w, ff_w, ffb2)


def kernel(sequence, group_id, table, conv_w, conv_b, proj_w, proj_b,
           score_w, score_b, ff_w, ff_b):
    B, S, D = _B, _S, _D
    del score_b
    seq3 = sequence.reshape(B, 1, S)
    convw2 = conv_w[:, 0, :].T.reshape(_NGRAM, D)
    convb2 = conv_b.reshape(1, D)
    projb2 = proj_b.reshape(1, D)
    ffb2 = ff_b.reshape(1, D)

    embed = _tc_pre(seq3, table, convw2, convb2, proj_w, projb2)

    gmax = group_id[:, :, -1:]
    gadj = jnp.where(group_id == 0, gmax, group_id - 1)
    idx3 = (gadj + (jnp.arange(_NL, dtype=jnp.int32) * _NSEG)[None, :, None])
    idx3 = idx3.reshape(B, _NL, _NC, _CH).astype(jnp.int32)
    zacc = jnp.zeros((_NL * _NSEG, _HW), jnp.float32)

    sums = _sc_segment_sums(embed, idx3, zacc)
    return _tc_post(seq3, group_id, sums, embed, score_w, ff_w, ffb2)


# speedup vs baseline: 1.0150x; 1.0150x over previous
import functools

import jax
import jax.numpy as jnp
from jax import lax
from jax.experimental import pallas as pl
from jax.experimental.pallas import tpu as pltpu
from jax.experimental.pallas import tpu_sc as plsc

_B, _S, _D, _V, _NGRAM = 8, 512, 256, 384, 4
_BLOCKS = _NGRAM * (_NGRAM + 1) // 2
_NL = _BLOCKS - 1
_NSEG = 64
_HW = 128
_CH = 128
_NC = _S // _CH
_NROW = _NL * _NSEG
_INTERPRET = False


def _pre_body(seq_ref, table_ref, convw_ref, convb_ref, projw_ref,
              projb_ref, emb_ref, nz_ref, ep_ref):
    f32 = jnp.float32
    S, D, V = _S, _D, _V
    seq_row = seq_ref[0].astype(f32)
    viota = lax.broadcasted_iota(jnp.int32, (V, S), 0).astype(f32)
    oht = (viota == seq_row).astype(f32)
    emb = lax.dot_general(oht, table_ref[...], (((0,), (0,)), ((), ())),
                          preferred_element_type=f32)
    nzv = (lax.broadcasted_iota(jnp.int32, (V, 8), 0) > 0).astype(f32)
    nz_col = lax.dot_general(oht, nzv, (((0,), (0,)), ((), ())),
                             preferred_element_type=f32)
    ep_ref[0:S, :] = emb
    ep_ref[S:S + 8, :] = jnp.zeros((8, D), f32)
    cw = convw_ref[...]
    acc = ep_ref[0:S, :] * cw[0:1, :]
    for tap in range(1, _NGRAM):
        acc += ep_ref[tap:tap + S, :] * cw[tap:tap + 1, :]
    acc += convb_ref[...]
    embed = lax.dot_general(acc, projw_ref[...], (((1,), (1,)), ((), ())),
                            preferred_element_type=f32) + projb_ref[...]
    emb_ref[0] = embed * nz_col[:, 0:1]
    nz_ref[0] = nz_col


def _tc_pre(seq3, table, convw2, convb2, proj_w, projb2):
    S, D, V = _S, _D, _V
    return pl.pallas_call(
        _pre_body,
        grid=(_B,),
        in_specs=[
            pl.BlockSpec((1, 1, S), lambda b: (b, 0, 0)),
            pl.BlockSpec((V, D), lambda b: (0, 0)),
            pl.BlockSpec((_NGRAM, D), lambda b: (0, 0)),
            pl.BlockSpec((1, D), lambda b: (0, 0)),
            pl.BlockSpec((D, D), lambda b: (0, 0)),
            pl.BlockSpec((1, D), lambda b: (0, 0)),
        ],
        out_specs=[
            pl.BlockSpec((1, S, D), lambda b: (b, 0, 0)),
            pl.BlockSpec((1, S, 8), lambda b: (b, 0, 0)),
        ],
        out_shape=[
            jax.ShapeDtypeStruct((_B, S, D), jnp.float32),
            jax.ShapeDtypeStruct((_B, S, 8), jnp.float32),
        ],
        scratch_shapes=[pltpu.VMEM((S + 8, D), jnp.float32)],
        compiler_params=pltpu.CompilerParams(
            dimension_semantics=("arbitrary",)),
        interpret=_INTERPRET,
    )(seq3, table, convw2, convb2, proj_w, projb2)


def _sc_segment_sums(embed, idx5):
    mesh = plsc.VectorSubcoreMesh(core_axis_name="c", subcore_axis_name="s")

    @functools.partial(
        pl.kernel, mesh=mesh,
        out_type=jax.ShapeDtypeStruct((_B, 2, _NROW, _HW), jnp.float32),
        scratch_types=[
            pltpu.VMEM((_NL, 2, _CH), jnp.int32),
            pltpu.VMEM_SHARED((_B * _NROW, _HW), jnp.float32),
            pltpu.VMEM((_CH, _HW), jnp.float32),
            pltpu.VMEM((96, _HW), jnp.float32),
            pltpu.SemaphoreType.DMA,
        ],
    )
    def k(e_hbm, idx_hbm, out_hbm, idx_v, acc_sh, ch_v, zb_v, sem):
        s = lax.axis_index("s")
        h = lax.axis_index("c")
        b = s // 2
        r = s % 2
        cp_idx = pltpu.async_copy(idx_hbm.at[b, r], idx_v, sem)
        for kk in range(96 * _HW // 16):
            zb_v[kk // (_HW // 16),
                 pl.ds((kk % (_HW // 16)) * 16, 16)] = jnp.zeros((16,),
                                                                 jnp.float32)
        for j in range(3):
            pltpu.sync_copy(zb_v,
                            acc_sh.at[pl.ds(b * _NROW + r * 288 + j * 96,
                                            96)])
        cp_idx.wait()
        plsc.subcore_barrier()
        cps = []
        for c in range(2):
            pltpu.sync_copy(
                e_hbm.at[b, pl.ds((2 * r + c) * _CH, _CH),
                         pl.ds(h * _HW, _HW)], ch_v)
            cps = [pltpu.async_copy(ch_v, acc_sh.at[idx_v.at[l, c]],
                                    sem, add=True) for l in range(_NL)]
            for cp in cps:
                cp.wait()
        plsc.subcore_barrier()
        pltpu.sync_copy(acc_sh.at[pl.ds(b * _NROW + r * 288, 288)],
                        out_hbm.at[b, h, pl.ds(r * 288, 288)])

    return k(embed, idx5)


def _post_body(gid_ref, nz_ref, sums_ref, emb_ref, scorew_ref, ffw_ref,
               ffb_ref, out_ref, reps_ref):
    f32 = jnp.float32
    S, D = _S, _D
    embed = emb_ref[0]
    reps_ref[0] = embed

    def block_score(rep):
        return lax.dot_general(rep, scorew_ref[...], (((1,), (1,)), ((), ())),
                               preferred_element_type=f32)

    scores = [(block_score(embed), nz_ref[0][:, 0:1] == 0.0)]

    giota = lax.broadcasted_iota(jnp.int32, (_NSEG, S), 0).astype(f32)
    ones_row = jnp.full((1, S), 1.0, f32)
    srows = lax.broadcasted_iota(jnp.int32, (S, _NSEG), 0).astype(f32)
    scol = lax.broadcasted_iota(jnp.int32, (S, 1), 0)
    tri = (lax.broadcasted_iota(jnp.int32, (_NSEG, _NSEG), 0)
           < lax.broadcasted_iota(jnp.int32, (_NSEG, _NSEG), 1)).astype(f32)
    gl_row = lax.broadcasted_iota(jnp.int32, (1, _NSEG), 1).astype(f32)
    gid_all = gid_ref[0]
    sums_all = sums_ref[0]
    for l in range(_NL):
        g_row = gid_all[l:l + 1, :].astype(f32)
        gmax = jnp.max(g_row)
        z = jnp.sum((g_row == 0.0).astype(jnp.int32))
        gadj_row = jnp.where(g_row == 0.0, gmax, g_row - 1.0)
        M2 = (gadj_row == giota).astype(f32)
        counts = lax.dot_general(ones_row, M2, (((1,), (1,)), ((), ())),
                                 preferred_element_type=f32)
        recip = ((1.0 / jnp.maximum(counts, 1.0))
                 * (gl_row != gmax).astype(f32))
        cum = jnp.dot(counts, tri, preferred_element_type=f32)
        G = ((srows >= cum) & (srows < cum + counts)).astype(f32) * recip
        rsl = slice(l * _NSEG, (l + 1) * _NSEG)
        seg = jnp.concatenate([sums_all[0, rsl, :], sums_all[1, rsl, :]],
                              axis=1)
        rep = jnp.dot(G, seg, preferred_element_type=f32)
        reps_ref[l + 1] = rep
        scores.append((block_score(rep), scol < z))

    neg = -jnp.finfo(f32).max
    svals = [jnp.where(m, neg, s) for s, m in scores]
    mval = svals[0]
    for s in svals[1:]:
        mval = jnp.maximum(mval, s)
    exps = [jnp.exp(s - mval) for s in svals]
    den = exps[0]
    for e_ in exps[1:]:
        den = den + e_
    out = reps_ref[0] * (exps[0] / den)
    for l in range(1, _BLOCKS):
        out += reps_ref[l] * (exps[l] / den)
    y = lax.dot_general(out, ffw_ref[...], (((1,), (1,)), ((), ())),
                        preferred_element_type=f32) + ffb_ref[...]
    out_ref[0] = out + jnp.maximum(y, 0.0)


def _tc_post(group_id, nzmask, sums, embed, score_w, ff_w, ffb2):
    S, D = _S, _D
    return pl.pallas_call(
        _post_body,
        grid=(_B,),
        in_specs=[
            pl.BlockSpec((1, _NL, S), lambda b: (b, 0, 0)),
            pl.BlockSpec((1, S, 8), lambda b: (b, 0, 0)),
            pl.BlockSpec((1, 2, _NROW, _HW), lambda b: (b, 0, 0, 0)),
            pl.BlockSpec((1, S, D), lambda b: (b, 0, 0)),
            pl.BlockSpec((1, D), lambda b: (0, 0)),
            pl.BlockSpec((D, D), lambda b: (0, 0)),
            pl.BlockSpec((1, D), lambda b: (0, 0)),
        ],
        out_specs=pl.BlockSpec((1, S, D), lambda b: (b, 0, 0)),
        out_shape=jax.ShapeDtypeStruct((_B, S, D), jnp.float32),
        scratch_shapes=[pltpu.VMEM((_BLOCKS, S, D), jnp.float32)],
        compiler_params=pltpu.CompilerParams(
            dimension_semantics=("arbitrary",)),
        interpret=_INTERPRET,
    )(group_id, nzmask, sums, embed, score_w, ff_w, ffb2)


def kernel(sequence, group_id, table, conv_w, conv_b, proj_w, proj_b,
           score_w, score_b, ff_w, ff_b):
    B, S, D = _B, _S, _D
    del score_b
    seq3 = sequence.reshape(B, 1, S)
    convw2 = conv_w[:, 0, :].T.reshape(_NGRAM, D)
    convb2 = conv_b.reshape(1, D)
    projb2 = proj_b.reshape(1, D)
    ffb2 = ff_b.reshape(1, D)

    embed, nzmask = _tc_pre(seq3, table, convw2, convb2, proj_w, projb2)

    gmax = group_id[:, :, -1:]
    gadj = jnp.where(group_id == 0, gmax, group_id - 1)
    idx = (gadj
           + (jnp.arange(_NL, dtype=jnp.int32) * _NSEG)[None, :, None]
           + (jnp.arange(_B, dtype=jnp.int32) * _NROW)[:, None, None])
    idx5 = jnp.transpose(idx.reshape(B, _NL, 2, 2, _CH), (0, 2, 1, 3, 4))
    idx5 = idx5.astype(jnp.int32)

    sums = _sc_segment_sums(embed, idx5)
    return _tc_post(group_id, nzmask, sums, embed, score_w, ff_w, ffb2)
